# baseline (speedup 1.0000x reference)
import functools

import jax
import jax.numpy as jnp
from jax import lax
from jax.experimental import pallas as pl
from jax.experimental.pallas import tpu as pltpu
from jax.experimental.pallas import tpu_sc as plsc

_NC = 2
_NS = 16
_CHUNK = 128
_NBUF = 6
_BLK = 2000


def _sc_mesh():
    return plsc.VectorSubcoreMesh(core_axis_name="c", subcore_axis_name="s")


def _make_deg_kernel(n_acc, cpt, dd):
    rpt = n_acc // _NS

    @functools.partial(
        pl.kernel,
        out_type=jax.ShapeDtypeStruct((_NC, n_acc, dd), jnp.float32),
        mesh=_sc_mesh(),
        compiler_params=pltpu.CompilerParams(use_tc_tiling_on_sc=False),
        scratch_types=[
            pltpu.VMEM((cpt, _CHUNK), jnp.int32),
            pltpu.VMEM((_CHUNK, dd), jnp.float32),
            pltpu.VMEM_SHARED((n_acc, dd), jnp.float32),
        ],
    )
    def deg_kernel(dst_hbm, ones_hbm, zeros_hbm, out_hbm,
                   dst_v, ones_v, acc_sh):
        c = lax.axis_index("c")
        s = lax.axis_index("s")
        pltpu.sync_copy(zeros_hbm, acc_sh.at[pl.ds(s * rpt, rpt)])
        pltpu.sync_copy(ones_hbm, ones_v)
        pltpu.sync_copy(dst_hbm.at[s], dst_v)
        plsc.subcore_barrier()

        def body(j, carry):
            jc = 2 * j + c

            @pl.when(jc < cpt)
            def _():
                pltpu.sync_copy(ones_v, acc_sh.at[dst_v.at[jc]], add=True)

            return carry

        lax.fori_loop(0, -(-cpt // 2), body, 0)
        plsc.subcore_barrier()
        pltpu.sync_copy(acc_sh.at[pl.ds(s * rpt, rpt)],
                        out_hbm.at[c, pl.ds(s * rpt, rpt)])

    return deg_kernel


def _make_agg_kernel(n_acc, cpt, dh, alt=False):
    rpt = n_acc // _NS

    @functools.partial(
        pl.kernel,
        out_type=jax.ShapeDtypeStruct((_NC, n_acc, dh), jnp.float32),
        mesh=_sc_mesh(),
        compiler_params=pltpu.CompilerParams(use_tc_tiling_on_sc=False),
        scratch_types=[
            pltpu.VMEM((cpt, _CHUNK), jnp.int32),
            pltpu.VMEM((cpt, _CHUNK), jnp.int32),
            pltpu.VMEM((_NBUF, _CHUNK, dh), jnp.float32),
            pltpu.VMEM_SHARED((n_acc, dh), jnp.float32),
        ] + [pltpu.SemaphoreType.DMA] * _NBUF,
    )
    def agg_kernel(src_hbm, dst_hbm, tbl_hbm, zeros_hbm, out_hbm,
                   src_v, dst_v, rows_v, acc_sh, *gsem):
        c = lax.axis_index("c")
        s = lax.axis_index("s")
        tbl = tbl_hbm if alt else tbl_hbm.at[c]
        cidx = (lambda k: 2 * k + c) if alt else (lambda k: k)
        nk = -(-cpt // 2) if alt else cpt
        pltpu.sync_copy(zeros_hbm, acc_sh.at[pl.ds(s * rpt, rpt)])
        pltpu.sync_copy(src_hbm.at[s], src_v)
        pltpu.sync_copy(dst_hbm.at[s], dst_v)
        plsc.subcore_barrier()

        for b in range(_NBUF):
            jc = cidx(b)
            if alt:
                @pl.when(jc < cpt)
                def _():
                    pltpu.async_copy(tbl.at[src_v.at[jc]], rows_v.at[b],
                                     gsem[b])
            else:
                pltpu.async_copy(tbl.at[src_v.at[jc]], rows_v.at[b], gsem[b])

        def group(gi, carry):
            for b in range(_NBUF):
                k = gi * _NBUF + b
                jc = cidx(k)

                @pl.when(jc < cpt)
                def _():
                    pltpu.make_async_copy(tbl.at[src_v.at[jc]], rows_v.at[b],
                                          gsem[b]).wait()
                    pltpu.sync_copy(rows_v.at[b], acc_sh.at[dst_v.at[jc]],
                                    add=True)
                    jn = cidx(k + _NBUF)

                    @pl.when(jn < cpt)
                    def _():
                        pltpu.async_copy(tbl.at[src_v.at[jn]],
                                         rows_v.at[b], gsem[b])

            return carry

        lax.fori_loop(0, -(-nk // _NBUF), group, 0)
        plsc.subcore_barrier()
        pltpu.sync_copy(acc_sh.at[pl.ds(s * rpt, rpt)],
                        out_hbm.at[c, pl.ds(s * rpt, rpt)])

    return agg_kernel


def _tc_layer1(x, w1, degp, n_acc):
    n, f_in = x.shape
    f_hid = w1.shape[1]
    dh = f_hid // 2
    grid = n // _BLK

    def body(x_ref, w_ref, degp_ref, h_ref, dinv_ref):
        deg = degp_ref[0, :, 0] + degp_ref[1, :, 0]
        dinv = jnp.where(deg > 0.0, lax.rsqrt(jnp.maximum(deg, 1e-12)), 0.0)
        h = jnp.dot(x_ref[...], w_ref[...], preferred_element_type=jnp.float32)
        h = h * dinv[:, None]
        h_ref[0] = h[:, :dh]
        h_ref[1] = h[:, dh:]
        dinv_ref[...] = jnp.broadcast_to(dinv[:, None], dinv_ref.shape)

    return pl.pallas_call(
        body,
        grid=(grid,),
        in_specs=[
            pl.BlockSpec((_BLK, f_in), lambda i: (i, 0)),
            pl.BlockSpec((f_in, f_hid), lambda i: (0, 0)),
            pl.BlockSpec((2, _BLK, 8), lambda i: (0, i, 0)),
        ],
        out_specs=[
            pl.BlockSpec((2, _BLK, dh), lambda i: (0, i, 0)),
            pl.BlockSpec((_BLK, 16), lambda i: (i, 0)),
        ],
        out_shape=[jax.ShapeDtypeStruct((2, n_acc, dh), jnp.float32),
                   jax.ShapeDtypeStruct((n, 16), jnp.float32)],
    )(x, w1, degp)


def _tc_mid(aggp, dinvb, b1, w2, n):
    n_acc = aggp.shape[1]
    dh = aggp.shape[2]
    f_hid = 2 * dh
    f_out = w2.shape[1]
    do = f_out // 2
    grid = n // _BLK

    def body(aggp_ref, dinv_ref, b1_ref, w2_ref, out_ref):
        agg = jnp.concatenate([aggp_ref[0], aggp_ref[1]], axis=1)
        dinv = dinv_ref[...][:, :1]
        hmid = jnp.maximum(agg * dinv + b1_ref[...], 0.0)
        h2 = jnp.dot(hmid, w2_ref[...], preferred_element_type=jnp.float32)
        out_ref[...] = h2 * dinv

    return pl.pallas_call(
        body,
        grid=(grid,),
        in_specs=[
            pl.BlockSpec((2, _BLK, dh), lambda i: (0, i, 0)),
            pl.BlockSpec((_BLK, 16), lambda i: (i, 0)),
            pl.BlockSpec((1, f_hid), lambda i: (0, 0)),
            pl.BlockSpec((f_hid, f_out), lambda i: (0, 0)),
        ],
        out_specs=pl.BlockSpec((_BLK, f_out), lambda i: (i, 0)),
        out_shape=jax.ShapeDtypeStruct((n_acc, f_out), jnp.float32),
    )(aggp, dinvb, b1, w2)


def _tc_final(aggp, dinvb, b2, n):
    f_out = aggp.shape[2]
    grid = n // _BLK

    def body(aggp_ref, dinv_ref, b2_ref, out_ref):
        agg = aggp_ref[0] + aggp_ref[1]
        dinv = dinv_ref[...][:, :1]
        o = agg * dinv + b2_ref[...]
        m = jnp.max(o, axis=1, keepdims=True)
        ex = jnp.exp(o - m)
        lse = jnp.log(jnp.sum(ex, axis=1, keepdims=True)) + m
        out_ref[...] = o - lse

    return pl.pallas_call(
        body,
        grid=(grid,),
        in_specs=[
            pl.BlockSpec((2, _BLK, f_out), lambda i: (0, i, 0)),
            pl.BlockSpec((_BLK, 16), lambda i: (i, 0)),
            pl.BlockSpec((1, f_out), lambda i: (0, 0)),
        ],
        out_specs=pl.BlockSpec((_BLK, f_out), lambda i: (i, 0)),
        out_shape=jax.ShapeDtypeStruct((n, f_out), jnp.float32),
    )(aggp, dinvb, b2)


def kernel(x, edge_index, W1, b1, W2, b2):
    n, f_in = x.shape
    e = edge_index.shape[1]
    f_hid = W1.shape[1]
    f_out = W2.shape[1]

    cpt = -(-e // (_NS * _CHUNK))
    n_acc = -(-(n + 1) // (_NS * 8)) * (_NS * 8)
    rpt = n_acc // _NS
    dh = f_hid // 2
    do = f_out // 2

    pad = _NS * cpt * _CHUNK - e
    ei = jnp.pad(edge_index, ((0, 0), (0, pad)), constant_values=n)
    src_q = ei[0].reshape(_NS, cpt, _CHUNK)
    dst_q = ei[1].reshape(_NS, cpt, _CHUNK)

    ones_d = jnp.ones((_CHUNK, do), jnp.float32)
    zeros_do = jnp.zeros((rpt, do), jnp.float32)
    zeros_dh = jnp.zeros((rpt, dh), jnp.float32)
    zeros_fo = jnp.zeros((rpt, f_out), jnp.float32)

    degp = _make_deg_kernel(n_acc, cpt, do)(dst_q, ones_d, zeros_do)
    h1s, dinvb = _tc_layer1(x, W1, degp, n_acc)
    agg1s = _make_agg_kernel(n_acc, cpt, dh)(src_q, dst_q, h1s, zeros_dh)
    h2s = _tc_mid(agg1s, dinvb, b1.reshape(1, f_hid), W2, n)
    agg2s = _make_agg_kernel(n_acc, cpt, f_out, alt=True)(src_q, dst_q,
                                                          h2s, zeros_fo)
    return _tc_final(agg2s, dinvb, b2.reshape(1, f_out), n)


# speedup vs baseline: 1.0229x; 1.0229x over previous
import functools

import jax
import jax.numpy as jnp
from jax import lax
from jax.experimental import pallas as pl
from jax.experimental.pallas import tpu as pltpu
from jax.experimental.pallas import tpu_sc as plsc

_NC = 2
_NS = 16
_CHUNK = 128
_NBUF = 6
_BLK = 2000


def _sc_mesh():
    return plsc.VectorSubcoreMesh(core_axis_name="c", subcore_axis_name="s")


def _make_deg_kernel(n_acc, cpt, dd):
    rpt = n_acc // _NS

    @functools.partial(
        pl.kernel,
        out_type=jax.ShapeDtypeStruct((_NC, n_acc, dd), jnp.float32),
        mesh=_sc_mesh(),
        compiler_params=pltpu.CompilerParams(use_tc_tiling_on_sc=False),
        scratch_types=[
            pltpu.VMEM((cpt, _CHUNK), jnp.int32),
            pltpu.VMEM((_CHUNK, dd), jnp.float32),
            pltpu.VMEM_SHARED((n_acc, dd), jnp.float32),
        ],
    )
    def deg_kernel(dst_hbm, ones_hbm, zeros_hbm, out_hbm,
                   dst_v, ones_v, acc_sh):
        c = lax.axis_index("c")
        s = lax.axis_index("s")
        pltpu.sync_copy(zeros_hbm, acc_sh.at[pl.ds(s * rpt, rpt)])
        pltpu.sync_copy(ones_hbm, ones_v)
        pltpu.sync_copy(dst_hbm.at[s], dst_v)
        plsc.subcore_barrier()

        def body(j, carry):
            jc = 2 * j + c

            @pl.when(jc < cpt)
            def _():
                pltpu.sync_copy(ones_v, acc_sh.at[dst_v.at[jc]], add=True)

            return carry

        lax.fori_loop(0, -(-cpt // 2), body, 0)
        plsc.subcore_barrier()
        pltpu.sync_copy(acc_sh.at[pl.ds(s * rpt, rpt)],
                        out_hbm.at[c, pl.ds(s * rpt, rpt)])

    return deg_kernel


def _make_agg_kernel(n_acc, cpt, dh, alt=False):
    rpt = n_acc // _NS

    @functools.partial(
        pl.kernel,
        out_type=jax.ShapeDtypeStruct((_NC, n_acc, dh), jnp.float32),
        mesh=_sc_mesh(),
        compiler_params=pltpu.CompilerParams(use_tc_tiling_on_sc=False),
        scratch_types=[
            pltpu.VMEM((cpt, _CHUNK), jnp.int32),
            pltpu.VMEM((cpt, _CHUNK), jnp.int32),
            pltpu.VMEM((_NBUF, _CHUNK, dh), jnp.float32),
            pltpu.VMEM_SHARED((n_acc, dh), jnp.float32),
        ] + [pltpu.SemaphoreType.DMA] * _NBUF,
    )
    def agg_kernel(src_hbm, dst_hbm, tbl_hbm, zeros_hbm, out_hbm,
                   src_v, dst_v, rows_v, acc_sh, *gsem):
        c = lax.axis_index("c")
        s = lax.axis_index("s")
        tbl = tbl_hbm if alt else tbl_hbm.at[c]
        cidx = (lambda k: 2 * k + c) if alt else (lambda k: k)
        nk = -(-cpt // 2) if alt else cpt
        pltpu.sync_copy(zeros_hbm, acc_sh.at[pl.ds(s * rpt, rpt)])
        pltpu.sync_copy(src_hbm.at[s], src_v)
        pltpu.sync_copy(dst_hbm.at[s], dst_v)
        plsc.subcore_barrier()

        for b in range(_NBUF):
            jc = cidx(b)
            if alt:
                @pl.when(jc < cpt)
                def _():
                    pltpu.async_copy(tbl.at[src_v.at[jc]], rows_v.at[b],
                                     gsem[b])
            else:
                pltpu.async_copy(tbl.at[src_v.at[jc]], rows_v.at[b], gsem[b])

        def group(gi, carry):
            for b in range(_NBUF):
                k = gi * _NBUF + b
                jc = cidx(k)

                @pl.when(jc < cpt)
                def _():
                    pltpu.make_async_copy(tbl.at[src_v.at[jc]], rows_v.at[b],
                                          gsem[b]).wait()
                    pltpu.sync_copy(rows_v.at[b], acc_sh.at[dst_v.at[jc]],
                                    add=True)
                    jn = cidx(k + _NBUF)

                    @pl.when(jn < cpt)
                    def _():
                        pltpu.async_copy(tbl.at[src_v.at[jn]],
                                         rows_v.at[b], gsem[b])

            return carry

        lax.fori_loop(0, -(-nk // _NBUF), group, 0)
        plsc.subcore_barrier()
        pltpu.sync_copy(acc_sh.at[pl.ds(s * rpt, rpt)],
                        out_hbm.at[c, pl.ds(s * rpt, rpt)])

    return agg_kernel


def _tc_layer1(x, w1, degp, n_acc):
    n, f_in = x.shape
    f_hid = w1.shape[1]
    dh = f_hid // 2
    grid = n // _BLK

    def body(x_ref, w_ref, degp_ref, h_ref, dinv_ref):
        deg = degp_ref[0, :, 0] + degp_ref[1, :, 0]
        dinv = jnp.where(deg > 0.0, lax.rsqrt(jnp.maximum(deg, 1e-12)), 0.0)
        h = jnp.dot(x_ref[...], w_ref[...], preferred_element_type=jnp.float32)
        h = h * dinv[:, None]
        h_ref[0] = h[:, :dh]
        h_ref[1] = h[:, dh:]
        dinv_ref[...] = jnp.broadcast_to(dinv[:, None], dinv_ref.shape)

    return pl.pallas_call(
        body,
        grid=(grid,),
        in_specs=[
            pl.BlockSpec((_BLK, f_in), lambda i: (i, 0)),
            pl.BlockSpec((f_in, f_hid), lambda i: (0, 0)),
            pl.BlockSpec((2, _BLK, 8), lambda i: (0, i, 0)),
        ],
        out_specs=[
            pl.BlockSpec((2, _BLK, dh), lambda i: (0, i, 0)),
            pl.BlockSpec((_BLK, 16), lambda i: (i, 0)),
        ],
        out_shape=[jax.ShapeDtypeStruct((2, n_acc, dh), jnp.float32),
                   jax.ShapeDtypeStruct((n, 16), jnp.float32)],
    )(x, w1, degp)


def _tc_mid(aggp, dinvb, b1, w2, n):
    n_acc = aggp.shape[1]
    dh = aggp.shape[2]
    f_hid = 2 * dh
    f_out = w2.shape[1]
    do = f_out // 2
    grid = n // _BLK

    def body(aggp_ref, dinv_ref, b1_ref, w2_ref, out_ref):
        agg = jnp.concatenate([aggp_ref[0], aggp_ref[1]], axis=1)
        dinv = dinv_ref[...][:, :1]
        hmid = jnp.maximum(agg * dinv + b1_ref[...], 0.0)
        h2 = jnp.dot(hmid, w2_ref[...], preferred_element_type=jnp.float32)
        h2 = h2 * dinv
        out_ref[0] = h2[:, :do]
        out_ref[1] = h2[:, do:]

    return pl.pallas_call(
        body,
        grid=(grid,),
        in_specs=[
            pl.BlockSpec((2, _BLK, dh), lambda i: (0, i, 0)),
            pl.BlockSpec((_BLK, 16), lambda i: (i, 0)),
            pl.BlockSpec((1, f_hid), lambda i: (0, 0)),
            pl.BlockSpec((f_hid, f_out), lambda i: (0, 0)),
        ],
        out_specs=pl.BlockSpec((2, _BLK, do), lambda i: (0, i, 0)),
        out_shape=jax.ShapeDtypeStruct((2, n_acc, do), jnp.float32),
    )(aggp, dinvb, b1, w2)


def _tc_final(aggp, dinvb, b2, n):
    do = aggp.shape[2]
    f_out = 2 * do
    grid = n // _BLK

    def body(aggp_ref, dinv_ref, b2_ref, out_ref):
        agg = jnp.concatenate([aggp_ref[0], aggp_ref[1]], axis=1)
        dinv = dinv_ref[...][:, :1]
        o = agg * dinv + b2_ref[...]
        m = jnp.max(o, axis=1, keepdims=True)
        ex = jnp.exp(o - m)
        lse = jnp.log(jnp.sum(ex, axis=1, keepdims=True)) + m
        out_ref[...] = o - lse

    return pl.pallas_call(
        body,
        grid=(grid,),
        in_specs=[
            pl.BlockSpec((2, _BLK, do), lambda i: (0, i, 0)),
            pl.BlockSpec((_BLK, 16), lambda i: (i, 0)),
            pl.BlockSpec((1, f_out), lambda i: (0, 0)),
        ],
        out_specs=pl.BlockSpec((_BLK, f_out), lambda i: (i, 0)),
        out_shape=jax.ShapeDtypeStruct((n, f_out), jnp.float32),
    )(aggp, dinvb, b2)


def kernel(x, edge_index, W1, b1, W2, b2):
    n, f_in = x.shape
    e = edge_index.shape[1]
    f_hid = W1.shape[1]
    f_out = W2.shape[1]

    cpt = -(-e // (_NS * _CHUNK))
    n_acc = -(-(n + 1) // (_NS * 8)) * (_NS * 8)
    rpt = n_acc // _NS
    dh = f_hid // 2
    do = f_out // 2

    pad = _NS * cpt * _CHUNK - e
    ei = jnp.pad(edge_index, ((0, 0), (0, pad)), constant_values=n)
    src_q = ei[0].reshape(_NS, cpt, _CHUNK)
    dst_q = ei[1].reshape(_NS, cpt, _CHUNK)

    ones_d = jnp.ones((_CHUNK, do), jnp.float32)
    zeros_do = jnp.zeros((rpt, do), jnp.float32)
    zeros_dh = jnp.zeros((rpt, dh), jnp.float32)
    zeros_fo = jnp.zeros((rpt, f_out), jnp.float32)

    degp = _make_deg_kernel(n_acc, cpt, do)(dst_q, ones_d, zeros_do)
    h1s, dinvb = _tc_layer1(x, W1, degp, n_acc)
    agg1s = _make_agg_kernel(n_acc, cpt, dh)(src_q, dst_q, h1s, zeros_dh)
    h2s = _tc_mid(agg1s, dinvb, b1.reshape(1, f_hid), W2, n)
    agg2s = _make_agg_kernel(n_acc, cpt, do)(src_q, dst_q, h2s, zeros_do)
    return _tc_final(agg2s, dinvb, b2.reshape(1, f_out), n)
